# Initial kernel scaffold; baseline (speedup 1.0000x reference)
import jax
import jax.numpy as jnp
from jax.experimental import pallas as pl


def kernel(x, edge_index, W1_l, W1_r, b1, W2_l, W2_r, b2):
    raise NotImplementedError("write your pallas kernel here")


# speedup vs baseline: 3.5635x; 3.5635x over previous
import functools
import math

import jax
import jax.numpy as jnp
from jax import lax
from jax.experimental import pallas as pl
from jax.experimental.pallas import tpu as pltpu
from jax.experimental.pallas import tpu_sc as plsc

_NC = 2
_NS = 16
_L = 16
_NW = _NC * _NS
_CB = 128

_F32 = jnp.float32


def _dot_t(a, w):
    return lax.dot_general(a, w, (((1,), (1,)), ((), ())),
                           preferred_element_type=_F32,
                           precision=lax.Precision.HIGHEST)


def _lin2_kernel(x_ref, wl_ref, wr_ref, b_ref, yl_ref, yr_ref):
    xb = x_ref[...]
    yl_ref[...] = _dot_t(xb, wl_ref[...])
    yr_ref[...] = _dot_t(xb, wr_ref[...]) + b_ref[...]


def _inv_kernel(cnt_ref, inv_ref):
    c = jnp.sum(cnt_ref[...], axis=0)
    inv_ref[...] = (1.0 / jnp.maximum(c, 1.0))[:, None]


def _mid_kernel(p0_ref, p1_ref, inv_ref, yr_ref, wl_ref, wr_ref, b_ref,
                y2l_ref, y2r_ref):
    h = (p0_ref[0] + p1_ref[0]) * inv_ref[...] + yr_ref[...]
    h = jnp.maximum(h, 0.0)
    y2l_ref[...] = _dot_t(h, wl_ref[...])
    y2r_ref[...] = _dot_t(h, wr_ref[...]) + b_ref[...]


def _final_kernel(q0_ref, q1_ref, inv_ref, yr_ref, out_ref):
    out_ref[...] = (q0_ref[0] + q1_ref[0]) * inv_ref[...] + yr_ref[...]


def _sc_aggregate(y, src_rs, dst_rs, n_acc, with_counts):
    n, d = y.shape
    ch = src_rs.shape[0] // _NW
    nzc = n_acc // _CB
    zc_max = -(-nzc // _NS)

    mesh = plsc.VectorSubcoreMesh(core_axis_name="c", subcore_axis_name="s",
                                  num_cores=_NC, num_subcores=_NS)
    out_type = (jax.ShapeDtypeStruct((_NC, n_acc, d), _F32),)
    if with_counts:
        out_type += (jax.ShapeDtypeStruct((_NW, n_acc), _F32),)
    scratch = [
        pltpu.VMEM_SHARED((n_acc, d), _F32),
        pltpu.VMEM((4, _CB), jnp.int32),
        pltpu.VMEM((4, _CB), jnp.int32),
        pltpu.VMEM((_CB, d), _F32),
        pltpu.VMEM((_CB, d), _F32),
        pltpu.VMEM((n_acc if with_counts else _L,), _F32),
        pltpu.SemaphoreType.DMA,
        pltpu.SemaphoreType.DMA,
        pltpu.SemaphoreType.DMA,
        pltpu.SemaphoreType.DMA,
        pltpu.SemaphoreType.DMA,
        pltpu.SemaphoreType.DMA,
        pltpu.SemaphoreType.DMA,
        pltpu.SemaphoreType.DMA,
    ]

    def body(y_hbm, src_hbm, dst_hbm, *rest):
        if with_counts:
            part_hbm, cnt_hbm = rest[0], rest[1]
            rest = rest[2:]
        else:
            part_hbm = rest[0]
            rest = rest[1:]
        (acc, src_i, dst_i, rows0, rows1, cnt_v,
         sg0, sg1, ss0, ss1, si0, si1, si2, si3) = rest
        cid = lax.axis_index("c")
        sid = lax.axis_index("s")
        wid = cid * _NS + sid
        zero = jnp.zeros((_L,), _F32)
        ones = jnp.ones((_L,), _F32)
        rows = (rows0, rows1)
        sg = (sg0, sg1)
        ss = (ss0, ss1)
        si = (si0, si1, si2, si3)
        base = wid * ch

        def stage_idx(g, s):
            pltpu.async_copy(src_hbm.at[base + g], src_i.at[s], si[s])
            pltpu.async_copy(dst_hbm.at[base + g], dst_i.at[s], si[s])

        def wait_idx(g, s):
            pltpu.make_async_copy(src_hbm.at[base + g], src_i.at[s],
                                  si[s]).wait()
            pltpu.make_async_copy(dst_hbm.at[base + g], dst_i.at[s],
                                  si[s]).wait()

        for s in range(4):
            stage_idx(s, s)

        def _zrow(i, _):
            for j in range(d // _L):
                rows0[i, pl.ds(j * _L, _L)] = zero
            return 0
        lax.fori_loop(0, _CB, _zrow, 0)
        for k in range(zc_max):
            c = sid + k * _NS
            if (k + 1) * _NS <= nzc:
                pltpu.sync_copy(rows0, acc.at[pl.ds(c * _CB, _CB)])
            else:
                @pl.when(c < nzc)
                def _():
                    pltpu.sync_copy(rows0, acc.at[pl.ds(c * _CB, _CB)])
        if with_counts:
            def _zcnt(i, _):
                cnt_v[pl.ds(i * _L, _L)] = zero
                return 0
            lax.fori_loop(0, n_acc // _L, _zcnt, 0)
        plsc.subcore_barrier()

        wait_idx(0, 0)
        pltpu.async_copy(y_hbm.at[src_i.at[0]], rows0, sg0)
        wait_idx(1, 1)
        pltpu.async_copy(y_hbm.at[src_i.at[1]], rows1, sg1)

        def _chunk(g, s, b):
            pltpu.make_async_copy(y_hbm.at[src_i.at[s]], rows[b],
                                  sg[b]).wait()
            sc_desc = pltpu.async_copy(rows[b], acc.at[dst_i.at[s]], ss[b],
                                       add=True)
            if with_counts:
                for j in range(_CB // _L):
                    ids = dst_i[s, pl.ds(j * _L, _L)]
                    plsc.addupdate_scatter(cnt_v, [ids], ones)
            sc_desc.wait()
            @pl.when(g + 4 < ch)
            def _():
                stage_idx(g + 4, s)
            @pl.when(g + 2 < ch)
            def _():
                s2 = (s + 2) % 4
                wait_idx(g + 2, s2)
                pltpu.async_copy(y_hbm.at[src_i.at[s2]], rows[b], sg[b])

        def _iter(i, _):
            g = 4 * i
            _chunk(g, 0, 0)
            _chunk(g + 1, 1, 1)
            _chunk(g + 2, 2, 0)
            _chunk(g + 3, 3, 1)
            return 0
        lax.fori_loop(0, ch // 4, _iter, 0)

        plsc.subcore_barrier()
        for k in range(zc_max):
            c = sid + k * _NS
            if (k + 1) * _NS <= nzc:
                pltpu.sync_copy(acc.at[pl.ds(c * _CB, _CB)],
                                part_hbm.at[cid, pl.ds(c * _CB, _CB)])
            else:
                @pl.when(c < nzc)
                def _():
                    pltpu.sync_copy(acc.at[pl.ds(c * _CB, _CB)],
                                    part_hbm.at[cid, pl.ds(c * _CB, _CB)])
        if with_counts:
            pltpu.sync_copy(cnt_v, cnt_hbm.at[wid])

    params = pltpu.CompilerParams(needs_layout_passes=False)
    return pl.kernel(body, out_type=out_type, mesh=mesh,
                     scratch_types=scratch,
                     compiler_params=params)(y, src_rs, dst_rs)


def kernel(x, edge_index, W1_l, W1_r, b1, W2_l, W2_r, b2):
    n, d = x.shape
    e = edge_index.shape[1]

    ch = -(-e // (_NW * _CB))
    ch = -(-ch // 4) * 4
    ep = _NW * _CB * ch
    n_acc = _CB * (-(-(n + 1) // _CB))

    src = edge_index[0].astype(jnp.int32)
    dst = edge_index[1].astype(jnp.int32)
    src_rs = jnp.concatenate(
        [src, jnp.zeros((ep - e,), jnp.int32)]).reshape(_NW * ch, _CB)
    dst_rs = jnp.concatenate(
        [dst, jnp.full((ep - e,), n, jnp.int32)]).reshape(_NW * ch, _CB)

    bn = max((b for b in range(8, min(n, 1024) + 1, 8) if n % b == 0),
             default=n)
    grid = (n // bn,)
    row_spec = pl.BlockSpec((bn, d), lambda i: (i, 0))
    w_spec = pl.BlockSpec((d, d), lambda i: (0, 0))
    b_spec = pl.BlockSpec((1, d), lambda i: (0, 0))
    part0_spec = pl.BlockSpec((1, bn, d), lambda i: (0, i, 0))
    part1_spec = pl.BlockSpec((1, bn, d), lambda i: (1, i, 0))
    inv_spec = pl.BlockSpec((bn, 1), lambda i: (i, 0))
    row_ty = jax.ShapeDtypeStruct((n, d), _F32)

    y1l, y1r = pl.pallas_call(
        _lin2_kernel, grid=grid,
        in_specs=[row_spec, w_spec, w_spec, b_spec],
        out_specs=[row_spec, row_spec],
        out_shape=[row_ty, row_ty],
    )(x, W1_l, W1_r, b1.reshape(1, d))

    part1, cnt = _sc_aggregate(y1l, src_rs, dst_rs, n_acc, True)

    inv = pl.pallas_call(
        _inv_kernel, grid=(n_acc // _CB,),
        in_specs=[pl.BlockSpec((_NW, _CB), lambda i: (0, i))],
        out_specs=pl.BlockSpec((_CB, 1), lambda i: (i, 0)),
        out_shape=jax.ShapeDtypeStruct((n_acc, 1), _F32),
    )(cnt)

    y2l, y2r = pl.pallas_call(
        _mid_kernel, grid=grid,
        in_specs=[part0_spec, part1_spec, inv_spec, row_spec,
                  w_spec, w_spec, b_spec],
        out_specs=[row_spec, row_spec],
        out_shape=[row_ty, row_ty],
    )(part1, part1, inv, y1r, W2_l, W2_r, b2.reshape(1, d))

    part2 = _sc_aggregate(y2l, src_rs, dst_rs, n_acc, False)[0]

    out = pl.pallas_call(
        _final_kernel, grid=grid,
        in_specs=[part0_spec, part1_spec, inv_spec, row_spec],
        out_specs=row_spec,
        out_shape=row_ty,
    )(part2, part2, inv, y2r)
    return out


# speedup vs baseline: 5.2886x; 1.4841x over previous
import functools
import math

import jax
import jax.numpy as jnp
from jax import lax
from jax.experimental import pallas as pl
from jax.experimental.pallas import tpu as pltpu
from jax.experimental.pallas import tpu_sc as plsc

_NC = 2
_NS = 16
_L = 16
_NW = _NC * _NS
_CB = 128

_F32 = jnp.float32


def _dot_t(a, w):
    return lax.dot_general(a, w, (((1,), (1,)), ((), ())),
                           preferred_element_type=_F32,
                           precision=lax.Precision.HIGHEST)


def _lin2_kernel(x_ref, wl_ref, wr_ref, b_ref, yl_ref, yr_ref):
    xb = x_ref[...]
    yl_ref[...] = _dot_t(xb, wl_ref[...]).astype(jnp.bfloat16)
    yr_ref[...] = _dot_t(xb, wr_ref[...]) + b_ref[...]


def _inv_kernel(cnt_ref, inv_ref):
    c = jnp.sum(cnt_ref[...], axis=0)
    inv_ref[...] = (1.0 / jnp.maximum(c, 1.0))[:, None]


def _mid_kernel(p0_ref, p1_ref, inv_ref, yr_ref, wl_ref, wr_ref, b_ref,
                y2l_ref, y2r_ref):
    h = (p0_ref[0] + p1_ref[0]) * inv_ref[...] + yr_ref[...]
    h = jnp.maximum(h, 0.0)
    y2l_ref[...] = _dot_t(h, wl_ref[...]).astype(jnp.bfloat16)
    y2r_ref[...] = _dot_t(h, wr_ref[...]) + b_ref[...]


def _final_kernel(q0_ref, q1_ref, inv_ref, yr_ref, out_ref):
    out_ref[...] = (q0_ref[0] + q1_ref[0]) * inv_ref[...] + yr_ref[...]


def _sc_aggregate(y, src_rs, dst_rs, n_acc, with_counts):
    n, d = y.shape
    ch = src_rs.shape[0] // _NW
    nzc = n_acc // _CB
    zc_max = -(-nzc // _NS)

    mesh = plsc.VectorSubcoreMesh(core_axis_name="c", subcore_axis_name="s",
                                  num_cores=_NC, num_subcores=_NS)
    out_type = (jax.ShapeDtypeStruct((_NC, n_acc, d), _F32),)
    if with_counts:
        out_type += (jax.ShapeDtypeStruct((_NW, n_acc), _F32),)
    scratch = [
        pltpu.VMEM_SHARED((n_acc, d), _F32),
        pltpu.VMEM((4, _CB), jnp.int32),
        pltpu.VMEM((4, _CB), jnp.int32),
        pltpu.VMEM((_CB, d), jnp.bfloat16),
        pltpu.VMEM((_CB, d), jnp.bfloat16),
        pltpu.VMEM((_CB, d), _F32),
        pltpu.VMEM((n_acc if with_counts else _L,), _F32),
        pltpu.SemaphoreType.DMA,
        pltpu.SemaphoreType.DMA,
        pltpu.SemaphoreType.DMA,
        pltpu.SemaphoreType.DMA,
        pltpu.SemaphoreType.DMA,
        pltpu.SemaphoreType.DMA,
        pltpu.SemaphoreType.DMA,
        pltpu.SemaphoreType.DMA,
    ]

    def body(y_hbm, src_hbm, dst_hbm, *rest):
        if with_counts:
            part_hbm, cnt_hbm = rest[0], rest[1]
            rest = rest[2:]
        else:
            part_hbm = rest[0]
            rest = rest[1:]
        (acc, src_i, dst_i, rows0, rows1, rowsf, cnt_v,
         sg0, sg1, ss0, ss1, si0, si1, si2, si3) = rest
        cid = lax.axis_index("c")
        sid = lax.axis_index("s")
        wid = cid * _NS + sid
        zero = jnp.zeros((_L,), _F32)
        ones = jnp.ones((_L,), _F32)
        rows = (rows0, rows1)
        sg = (sg0, sg1)
        ss = (ss0, ss1)
        si = (si0, si1, si2, si3)
        base = wid * ch

        def stage_idx(g, s):
            pltpu.async_copy(src_hbm.at[base + g], src_i.at[s], si[s])
            pltpu.async_copy(dst_hbm.at[base + g], dst_i.at[s], si[s])

        def wait_idx(g, s):
            pltpu.make_async_copy(src_hbm.at[base + g], src_i.at[s],
                                  si[s]).wait()
            pltpu.make_async_copy(dst_hbm.at[base + g], dst_i.at[s],
                                  si[s]).wait()

        for s in range(4):
            stage_idx(s, s)

        def _zrow(i, _):
            for j in range(d // _L):
                rowsf[i, pl.ds(j * _L, _L)] = zero
            return 0
        lax.fori_loop(0, _CB, _zrow, 0)
        for k in range(zc_max):
            c = sid + k * _NS
            if (k + 1) * _NS <= nzc:
                pltpu.sync_copy(rowsf, acc.at[pl.ds(c * _CB, _CB)])
            else:
                @pl.when(c < nzc)
                def _():
                    pltpu.sync_copy(rowsf, acc.at[pl.ds(c * _CB, _CB)])
        if with_counts:
            def _zcnt(i, _):
                cnt_v[pl.ds(i * _L, _L)] = zero
                return 0
            lax.fori_loop(0, n_acc // _L, _zcnt, 0)
        plsc.subcore_barrier()

        wait_idx(0, 0)
        pltpu.async_copy(y_hbm.at[src_i.at[0]], rows0, sg0)
        wait_idx(1, 1)
        pltpu.async_copy(y_hbm.at[src_i.at[1]], rows1, sg1)

        def _chunk(g, s, b):
            pltpu.make_async_copy(y_hbm.at[src_i.at[s]], rows[b],
                                  sg[b]).wait()
            def _conv(i, _):
                for j in range(d // (2 * _L)):
                    ab = rows[b][i, pl.ds(j * 2 * _L, 2 * _L)]
                    lo, hi = plsc.unpack(ab, format=plsc.PackFormat.INTERLEAVED)
                    rowsf[i, pl.ds(j * 2 * _L, _L)] = lo
                    rowsf[i, pl.ds(j * 2 * _L + _L, _L)] = hi
                return 0
            lax.fori_loop(0, _CB, _conv, 0)
            @pl.when(g + 2 < ch)
            def _():
                s2 = (s + 2) % 4
                wait_idx(g + 2, s2)
                pltpu.async_copy(y_hbm.at[src_i.at[s2]], rows[b], sg[b])
            sc_desc = pltpu.async_copy(rowsf, acc.at[dst_i.at[s]], ss[b],
                                       add=True)
            if with_counts:
                for j in range(_CB // _L):
                    ids = dst_i[s, pl.ds(j * _L, _L)]
                    plsc.addupdate_scatter(cnt_v, [ids], ones)
            sc_desc.wait()
            @pl.when(g + 4 < ch)
            def _():
                stage_idx(g + 4, s)

        def _iter(i, _):
            g = 4 * i
            _chunk(g, 0, 0)
            _chunk(g + 1, 1, 1)
            _chunk(g + 2, 2, 0)
            _chunk(g + 3, 3, 1)
            return 0
        lax.fori_loop(0, ch // 4, _iter, 0)

        plsc.subcore_barrier()
        for k in range(zc_max):
            c = sid + k * _NS
            if (k + 1) * _NS <= nzc:
                pltpu.sync_copy(acc.at[pl.ds(c * _CB, _CB)],
                                part_hbm.at[cid, pl.ds(c * _CB, _CB)])
            else:
                @pl.when(c < nzc)
                def _():
                    pltpu.sync_copy(acc.at[pl.ds(c * _CB, _CB)],
                                    part_hbm.at[cid, pl.ds(c * _CB, _CB)])
        if with_counts:
            pltpu.sync_copy(cnt_v, cnt_hbm.at[wid])

    params = pltpu.CompilerParams(needs_layout_passes=False,
                                  use_tc_tiling_on_sc=False)
    return pl.kernel(body, out_type=out_type, mesh=mesh,
                     scratch_types=scratch,
                     compiler_params=params)(y, src_rs, dst_rs)


def kernel(x, edge_index, W1_l, W1_r, b1, W2_l, W2_r, b2):
    n, d = x.shape
    e = edge_index.shape[1]

    ch = -(-e // (_NW * _CB))
    ch = -(-ch // 4) * 4
    ep = _NW * _CB * ch
    n_acc = _CB * (-(-(n + 1) // _CB))

    src = edge_index[0].astype(jnp.int32)
    dst = edge_index[1].astype(jnp.int32)
    src_rs = jnp.concatenate(
        [src, jnp.zeros((ep - e,), jnp.int32)]).reshape(_NW * ch, _CB)
    dst_rs = jnp.concatenate(
        [dst, jnp.full((ep - e,), n, jnp.int32)]).reshape(_NW * ch, _CB)

    colsrc = [0] * d
    for g in range(d // (2 * _L)):
        for j in range(_L):
            colsrc[g * 2 * _L + 2 * j] = g * 2 * _L + j
            colsrc[g * 2 * _L + 2 * j + 1] = g * 2 * _L + _L + j
    perm = jnp.array(colsrc, dtype=jnp.int32)
    W1_l_sc = W1_l[perm]
    W2_l_sc = W2_l[perm]

    bn = max((b for b in range(16, min(n, 1024) + 1, 16) if n % b == 0),
             default=n)
    grid = (n // bn,)
    row_spec = pl.BlockSpec((bn, d), lambda i: (i, 0))
    w_spec = pl.BlockSpec((d, d), lambda i: (0, 0))
    b_spec = pl.BlockSpec((1, d), lambda i: (0, 0))
    part0_spec = pl.BlockSpec((1, bn, d), lambda i: (0, i, 0))
    part1_spec = pl.BlockSpec((1, bn, d), lambda i: (1, i, 0))
    inv_spec = pl.BlockSpec((bn, 1), lambda i: (i, 0))
    row_ty = jax.ShapeDtypeStruct((n, d), _F32)
    tbl_ty = jax.ShapeDtypeStruct((n, d), jnp.bfloat16)

    y1l, y1r = pl.pallas_call(
        _lin2_kernel, grid=grid,
        in_specs=[row_spec, w_spec, w_spec, b_spec],
        out_specs=[row_spec, row_spec],
        out_shape=[tbl_ty, row_ty],
    )(x, W1_l_sc, W1_r, b1.reshape(1, d))

    part1, cnt = _sc_aggregate(y1l, src_rs, dst_rs, n_acc, True)

    inv = pl.pallas_call(
        _inv_kernel, grid=(n_acc // _CB,),
        in_specs=[pl.BlockSpec((_NW, _CB), lambda i: (0, i))],
        out_specs=pl.BlockSpec((_CB, 1), lambda i: (i, 0)),
        out_shape=jax.ShapeDtypeStruct((n_acc, 1), _F32),
    )(cnt)

    y2l, y2r = pl.pallas_call(
        _mid_kernel, grid=grid,
        in_specs=[part0_spec, part1_spec, inv_spec, row_spec,
                  w_spec, w_spec, b_spec],
        out_specs=[row_spec, row_spec],
        out_shape=[tbl_ty, row_ty],
    )(part1, part1, inv, y1r, W2_l_sc, W2_r, b2.reshape(1, d))

    part2 = _sc_aggregate(y2l, src_rs, dst_rs, n_acc, False)[0]

    out = pl.pallas_call(
        _final_kernel, grid=grid,
        in_specs=[part0_spec, part1_spec, inv_spec, row_spec],
        out_specs=row_spec,
        out_shape=row_ty,
    )(part2, part2, inv, y2r)
    return out


# speedup vs baseline: 5.3661x; 1.0146x over previous
import functools
import math

import jax
import jax.numpy as jnp
from jax import lax
from jax.experimental import pallas as pl
from jax.experimental.pallas import tpu as pltpu
from jax.experimental.pallas import tpu_sc as plsc

_NC = 2
_NS = 16
_L = 16
_NW = _NC * _NS
_CB = 128

_F32 = jnp.float32


def _dot_t(a, w):
    return lax.dot_general(a, w, (((1,), (1,)), ((), ())),
                           preferred_element_type=_F32,
                           precision=lax.Precision.HIGHEST)


def _lin2_kernel(x_ref, wl_ref, wr_ref, b_ref, yl_ref, yr_ref):
    xb = x_ref[...]
    yl_ref[...] = _dot_t(xb, wl_ref[...])
    yr_ref[...] = _dot_t(xb, wr_ref[...]) + b_ref[...]


def _inv_kernel(cnt_ref, inv_ref):
    c = jnp.sum(cnt_ref[...], axis=0)
    inv_ref[...] = (1.0 / jnp.maximum(c, 1.0))[:, None]


def _mid_kernel(p0_ref, p1_ref, inv_ref, yr_ref, wl_ref, wr_ref, b_ref,
                y2l_ref, y2r_ref):
    h = (p0_ref[0] + p1_ref[0]) * inv_ref[...] + yr_ref[...]
    h = jnp.maximum(h, 0.0)
    y2l_ref[...] = _dot_t(h, wl_ref[...])
    y2r_ref[...] = _dot_t(h, wr_ref[...]) + b_ref[...]


def _final_kernel(q0_ref, q1_ref, inv_ref, yr_ref, out_ref):
    out_ref[...] = (q0_ref[0] + q1_ref[0]) * inv_ref[...] + yr_ref[...]


def _sc_aggregate(y, src_rs, dst_rs, n_acc, with_counts):
    n, d = y.shape
    ch = src_rs.shape[0] // _NW
    nzc = n_acc // _CB
    zc_max = -(-nzc // _NS)

    mesh = plsc.VectorSubcoreMesh(core_axis_name="c", subcore_axis_name="s",
                                  num_cores=_NC, num_subcores=_NS)
    out_type = (jax.ShapeDtypeStruct((_NC, n_acc, d), _F32),)
    if with_counts:
        out_type += (jax.ShapeDtypeStruct((_NW, n_acc), _F32),)
    scratch = [
        pltpu.VMEM_SHARED((n_acc, d), _F32),
        pltpu.VMEM((4, _CB), jnp.int32),
        pltpu.VMEM((4, _CB), jnp.int32),
        pltpu.VMEM((_CB, d), jnp.bfloat16),
        pltpu.VMEM((_CB, d), jnp.bfloat16),
        pltpu.VMEM((_CB, d), _F32),
        pltpu.VMEM((n_acc if with_counts else _L,), _F32),
        pltpu.SemaphoreType.DMA,
        pltpu.SemaphoreType.DMA,
        pltpu.SemaphoreType.DMA,
        pltpu.SemaphoreType.DMA,
        pltpu.SemaphoreType.DMA,
        pltpu.SemaphoreType.DMA,
        pltpu.SemaphoreType.DMA,
        pltpu.SemaphoreType.DMA,
    ]

    def body(y_hbm, src_hbm, dst_hbm, *rest):
        if with_counts:
            part_hbm, cnt_hbm = rest[0], rest[1]
            rest = rest[2:]
        else:
            part_hbm = rest[0]
            rest = rest[1:]
        (acc, src_i, dst_i, rows0, rows1, rowsf, cnt_v,
         sg0, sg1, ss0, ss1, si0, si1, si2, si3) = rest
        cid = lax.axis_index("c")
        sid = lax.axis_index("s")
        wid = cid * _NS + sid
        zero = jnp.zeros((_L,), _F32)
        ones = jnp.ones((_L,), _F32)
        rows = (rows0, rows1)
        sg = (sg0, sg1)
        ss = (ss0, ss1)
        si = (si0, si1, si2, si3)
        base = wid * ch

        def stage_idx(g, s):
            pltpu.async_copy(src_hbm.at[base + g], src_i.at[s], si[s])
            pltpu.async_copy(dst_hbm.at[base + g], dst_i.at[s], si[s])

        def wait_idx(g, s):
            pltpu.make_async_copy(src_hbm.at[base + g], src_i.at[s],
                                  si[s]).wait()
            pltpu.make_async_copy(dst_hbm.at[base + g], dst_i.at[s],
                                  si[s]).wait()

        for s in range(4):
            stage_idx(s, s)

        def _zrow(i, _):
            for j in range(d // _L):
                rowsf[i, pl.ds(j * _L, _L)] = zero
            return 0
        lax.fori_loop(0, _CB, _zrow, 0)
        for k in range(zc_max):
            c = sid + k * _NS
            if (k + 1) * _NS <= nzc:
                pltpu.sync_copy(rowsf, acc.at[pl.ds(c * _CB, _CB)])
            else:
                @pl.when(c < nzc)
                def _():
                    pltpu.sync_copy(rowsf, acc.at[pl.ds(c * _CB, _CB)])
        if with_counts:
            def _zcnt(i, _):
                cnt_v[pl.ds(i * _L, _L)] = zero
                return 0
            lax.fori_loop(0, n_acc // _L, _zcnt, 0)
        plsc.subcore_barrier()

        wait_idx(0, 0)
        pltpu.async_copy(y_hbm.at[src_i.at[0]], rows0, sg0)
        wait_idx(1, 1)
        pltpu.async_copy(y_hbm.at[src_i.at[1]], rows1, sg1)

        def _chunk(g, s, b):
            pltpu.make_async_copy(y_hbm.at[src_i.at[s]], rows[b],
                                  sg[b]).wait()
            def _conv(i, _):
                for j in range(d // (2 * _L)):
                    ab = rows[b][i, pl.ds(j * 2 * _L, 2 * _L)]
                    lo, hi = plsc.unpack(ab, format=plsc.PackFormat.INTERLEAVED)
                    rowsf[i, pl.ds(j * 2 * _L, _L)] = lo
                    rowsf[i, pl.ds(j * 2 * _L + _L, _L)] = hi
                return 0
            lax.fori_loop(0, _CB, _conv, 0)
            @pl.when(g + 2 < ch)
            def _():
                s2 = (s + 2) % 4
                wait_idx(g + 2, s2)
                pltpu.async_copy(y_hbm.at[src_i.at[s2]], rows[b], sg[b])
            sc_desc = pltpu.async_copy(rowsf, acc.at[dst_i.at[s]], ss[b],
                                       add=True)
            if with_counts:
                for j in range(_CB // _L):
                    ids = dst_i[s, pl.ds(j * _L, _L)]
                    plsc.addupdate_scatter(cnt_v, [ids], ones)
            sc_desc.wait()
            @pl.when(g + 4 < ch)
            def _():
                stage_idx(g + 4, s)

        def _iter(i, _):
            g = 4 * i
            _chunk(g, 0, 0)
            _chunk(g + 1, 1, 1)
            _chunk(g + 2, 2, 0)
            _chunk(g + 3, 3, 1)
            return 0
        lax.fori_loop(0, ch // 4, _iter, 0)

        plsc.subcore_barrier()
        for k in range(zc_max):
            c = sid + k * _NS
            if (k + 1) * _NS <= nzc:
                pltpu.sync_copy(acc.at[pl.ds(c * _CB, _CB)],
                                part_hbm.at[cid, pl.ds(c * _CB, _CB)])
            else:
                @pl.when(c < nzc)
                def _():
                    pltpu.sync_copy(acc.at[pl.ds(c * _CB, _CB)],
                                    part_hbm.at[cid, pl.ds(c * _CB, _CB)])
        if with_counts:
            pltpu.sync_copy(cnt_v, cnt_hbm.at[wid])

    params = pltpu.CompilerParams(needs_layout_passes=False,
                                  use_tc_tiling_on_sc=False)
    return pl.kernel(body, out_type=out_type, mesh=mesh,
                     scratch_types=scratch,
                     compiler_params=params)(y, src_rs, dst_rs)


def kernel(x, edge_index, W1_l, W1_r, b1, W2_l, W2_r, b2):
    n, d = x.shape
    e = edge_index.shape[1]

    ch = -(-e // (_NW * _CB))
    ch = -(-ch // 4) * 4
    ep = _NW * _CB * ch
    n_acc = _CB * (-(-(n + 1) // _CB))

    src = edge_index[0].astype(jnp.int32)
    dst = edge_index[1].astype(jnp.int32)
    src_rs = jnp.concatenate(
        [src, jnp.zeros((ep - e,), jnp.int32)]).reshape(_NW * ch, _CB)
    dst_rs = jnp.concatenate(
        [dst, jnp.full((ep - e,), n, jnp.int32)]).reshape(_NW * ch, _CB)

    colsrc = [0] * d
    for g in range(d // (2 * _L)):
        for j in range(_L):
            colsrc[g * 2 * _L + 2 * j] = g * 2 * _L + j
            colsrc[g * 2 * _L + 2 * j + 1] = g * 2 * _L + _L + j
    perm = jnp.array(colsrc, dtype=jnp.int32)
    W1_l_sc = W1_l[perm]
    W2_l_sc = W2_l[perm]

    bn = max((b for b in range(16, min(n, 1024) + 1, 16) if n % b == 0),
             default=n)
    grid = (n // bn,)
    row_spec = pl.BlockSpec((bn, d), lambda i: (i, 0))
    w_spec = pl.BlockSpec((d, d), lambda i: (0, 0))
    b_spec = pl.BlockSpec((1, d), lambda i: (0, 0))
    part0_spec = pl.BlockSpec((1, bn, d), lambda i: (0, i, 0))
    part1_spec = pl.BlockSpec((1, bn, d), lambda i: (1, i, 0))
    inv_spec = pl.BlockSpec((bn, 1), lambda i: (i, 0))
    row_ty = jax.ShapeDtypeStruct((n, d), _F32)

    y1l, y1r = pl.pallas_call(
        _lin2_kernel, grid=grid,
        in_specs=[row_spec, w_spec, w_spec, b_spec],
        out_specs=[row_spec, row_spec],
        out_shape=[row_ty, row_ty],
    )(x, W1_l_sc, W1_r, b1.reshape(1, d))

    part1, cnt = _sc_aggregate(y1l.astype(jnp.bfloat16), src_rs, dst_rs,
                               n_acc, True)

    inv = pl.pallas_call(
        _inv_kernel, grid=(n_acc // _CB,),
        in_specs=[pl.BlockSpec((_NW, _CB), lambda i: (0, i))],
        out_specs=pl.BlockSpec((_CB, 1), lambda i: (i, 0)),
        out_shape=jax.ShapeDtypeStruct((n_acc, 1), _F32),
    )(cnt)

    y2l, y2r = pl.pallas_call(
        _mid_kernel, grid=grid,
        in_specs=[part0_spec, part1_spec, inv_spec, row_spec,
                  w_spec, w_spec, b_spec],
        out_specs=[row_spec, row_spec],
        out_shape=[row_ty, row_ty],
    )(part1, part1, inv, y1r, W2_l_sc, W2_r, b2.reshape(1, d))

    part2 = _sc_aggregate(y2l.astype(jnp.bfloat16), src_rs, dst_rs,
                          n_acc, False)[0]

    out = pl.pallas_call(
        _final_kernel, grid=grid,
        in_specs=[part0_spec, part1_spec, inv_spec, row_spec],
        out_specs=row_spec,
        out_shape=row_ty,
    )(part2, part2, inv, y2r)
    return out


# speedup vs baseline: 5.6799x; 1.0585x over previous
import functools
import math

import jax
import jax.numpy as jnp
from jax import lax
from jax.experimental import pallas as pl
from jax.experimental.pallas import tpu as pltpu
from jax.experimental.pallas import tpu_sc as plsc

_NC = 2
_NS = 16
_L = 16
_NW = _NC * _NS
_CB = 128

_F32 = jnp.float32


def _dot_t(a, w):
    return lax.dot_general(a, w, (((1,), (1,)), ((), ())),
                           preferred_element_type=_F32,
                           precision=lax.Precision.HIGHEST)


def _lin2_kernel(x_ref, wl_ref, wr_ref, b_ref, yl_ref, yr_ref):
    xb = x_ref[...]
    yl_ref[...] = _dot_t(xb, wl_ref[...])
    yr_ref[...] = _dot_t(xb, wr_ref[...]) + b_ref[...]


def _inv_kernel(cnt_ref, inv_ref):
    c = jnp.sum(cnt_ref[...], axis=0)
    inv_ref[...] = (1.0 / jnp.maximum(c, 1.0))[:, None]


def _mid_kernel(p0_ref, p1_ref, inv_ref, yr_ref, wl_ref, wr_ref, b_ref,
                y2l_ref, y2r_ref):
    h = (p0_ref[0] + p1_ref[0]) * inv_ref[...] + yr_ref[...]
    h = jnp.maximum(h, 0.0)
    y2l_ref[...] = _dot_t(h, wl_ref[...])
    y2r_ref[...] = _dot_t(h, wr_ref[...]) + b_ref[...]


def _final_kernel(q0_ref, q1_ref, inv_ref, yr_ref, out_ref):
    out_ref[...] = (q0_ref[0] + q1_ref[0]) * inv_ref[...] + yr_ref[...]


def _sc_aggregate(y, src_rs, dst_rs, n_acc, with_counts):
    n, d = y.shape
    ch = src_rs.shape[0] // _NW
    nzc = n_acc // _CB
    zc_max = -(-nzc // _NS)

    mesh = plsc.VectorSubcoreMesh(core_axis_name="c", subcore_axis_name="s",
                                  num_cores=_NC, num_subcores=_NS)
    out_type = (jax.ShapeDtypeStruct((_NC, n_acc, d), _F32),)
    if with_counts:
        out_type += (jax.ShapeDtypeStruct((_NW, n_acc), _F32),)
    scratch = [
        pltpu.VMEM_SHARED((n_acc, d), _F32),
        pltpu.VMEM((4, _CB), jnp.int32),
        pltpu.VMEM((4, _CB), jnp.int32),
        pltpu.VMEM((_CB, d), jnp.bfloat16),
        pltpu.VMEM((_CB, d), jnp.bfloat16),
        pltpu.VMEM((_CB, d), _F32),
        pltpu.VMEM((n_acc if with_counts else _L,), _F32),
        pltpu.SemaphoreType.DMA,
        pltpu.SemaphoreType.DMA,
        pltpu.SemaphoreType.DMA,
        pltpu.SemaphoreType.DMA,
        pltpu.SemaphoreType.DMA,
        pltpu.SemaphoreType.DMA,
        pltpu.SemaphoreType.DMA,
        pltpu.SemaphoreType.DMA,
    ]

    def body(y_hbm, src_hbm, dst_hbm, *rest):
        if with_counts:
            part_hbm, cnt_hbm = rest[0], rest[1]
            rest = rest[2:]
        else:
            part_hbm = rest[0]
            rest = rest[1:]
        (acc, src_i, dst_i, rows0, rows1, rowsf, cnt_v,
         sg0, sg1, ss0, ss1, si0, si1, si2, si3) = rest
        cid = lax.axis_index("c")
        sid = lax.axis_index("s")
        wid = cid * _NS + sid
        zero = jnp.zeros((_L,), _F32)
        ones = jnp.ones((_L,), _F32)
        rows = (rows0, rows1)
        sg = (sg0, sg1)
        ss = (ss0, ss1)
        si = (si0, si1, si2, si3)
        base = wid * ch

        def stage_idx(g, s):
            pltpu.async_copy(src_hbm.at[base + g], src_i.at[s], si[s])
            pltpu.async_copy(dst_hbm.at[base + g], dst_i.at[s], si[s])

        def wait_idx(g, s):
            pltpu.make_async_copy(src_hbm.at[base + g], src_i.at[s],
                                  si[s]).wait()
            pltpu.make_async_copy(dst_hbm.at[base + g], dst_i.at[s],
                                  si[s]).wait()

        for s in range(4):
            stage_idx(s, s)

        def _zrow(i, _):
            for j in range(d // _L):
                rowsf[i, pl.ds(j * _L, _L)] = zero
            return 0
        lax.fori_loop(0, _CB, _zrow, 0)
        for k in range(zc_max):
            c = sid + k * _NS
            if (k + 1) * _NS <= nzc:
                pltpu.sync_copy(rowsf, acc.at[pl.ds(c * _CB, _CB)])
            else:
                @pl.when(c < nzc)
                def _():
                    pltpu.sync_copy(rowsf, acc.at[pl.ds(c * _CB, _CB)])
        if with_counts:
            def _zcnt(i, _):
                cnt_v[pl.ds(i * _L, _L)] = zero
                return 0
            lax.fori_loop(0, n_acc // _L, _zcnt, 0)
        plsc.subcore_barrier()

        wait_idx(0, 0)
        pltpu.async_copy(y_hbm.at[src_i.at[0]], rows0, sg0)
        wait_idx(1, 1)
        pltpu.async_copy(y_hbm.at[src_i.at[1]], rows1, sg1)

        def _chunk(g, s, b):
            pltpu.make_async_copy(y_hbm.at[src_i.at[s]], rows[b],
                                  sg[b]).wait()
            @plsc.parallel_loop(0, _CB, unroll=4)
            def _conv(i):
                for j in range(d // (2 * _L)):
                    ab = rows[b][i, pl.ds(j * 2 * _L, 2 * _L)]
                    lo, hi = plsc.unpack(ab, format=plsc.PackFormat.INTERLEAVED)
                    rowsf[i, pl.ds(j * 2 * _L, _L)] = lo
                    rowsf[i, pl.ds(j * 2 * _L + _L, _L)] = hi
            @pl.when(g + 2 < ch)
            def _():
                s2 = (s + 2) % 4
                wait_idx(g + 2, s2)
                pltpu.async_copy(y_hbm.at[src_i.at[s2]], rows[b], sg[b])
            sc_desc = pltpu.async_copy(rowsf, acc.at[dst_i.at[s]], ss[b],
                                       add=True)
            if with_counts:
                for j in range(_CB // _L):
                    ids = dst_i[s, pl.ds(j * _L, _L)]
                    plsc.addupdate_scatter(cnt_v, [ids], ones)
            sc_desc.wait()
            @pl.when(g + 4 < ch)
            def _():
                stage_idx(g + 4, s)

        def _iter(i, _):
            g = 4 * i
            _chunk(g, 0, 0)
            _chunk(g + 1, 1, 1)
            _chunk(g + 2, 2, 0)
            _chunk(g + 3, 3, 1)
            return 0
        lax.fori_loop(0, ch // 4, _iter, 0)

        plsc.subcore_barrier()
        for k in range(zc_max):
            c = sid + k * _NS
            if (k + 1) * _NS <= nzc:
                pltpu.sync_copy(acc.at[pl.ds(c * _CB, _CB)],
                                part_hbm.at[cid, pl.ds(c * _CB, _CB)])
            else:
                @pl.when(c < nzc)
                def _():
                    pltpu.sync_copy(acc.at[pl.ds(c * _CB, _CB)],
                                    part_hbm.at[cid, pl.ds(c * _CB, _CB)])
        if with_counts:
            pltpu.sync_copy(cnt_v, cnt_hbm.at[wid])

    params = pltpu.CompilerParams(needs_layout_passes=False,
                                  use_tc_tiling_on_sc=False)
    return pl.kernel(body, out_type=out_type, mesh=mesh,
                     scratch_types=scratch,
                     compiler_params=params)(y, src_rs, dst_rs)


def kernel(x, edge_index, W1_l, W1_r, b1, W2_l, W2_r, b2):
    n, d = x.shape
    e = edge_index.shape[1]

    ch = -(-e // (_NW * _CB))
    ch = -(-ch // 4) * 4
    ep = _NW * _CB * ch
    n_acc = _CB * (-(-(n + 1) // _CB))

    src = edge_index[0].astype(jnp.int32)
    dst = edge_index[1].astype(jnp.int32)
    src_rs = jnp.concatenate(
        [src, jnp.zeros((ep - e,), jnp.int32)]).reshape(_NW * ch, _CB)
    dst_rs = jnp.concatenate(
        [dst, jnp.full((ep - e,), n, jnp.int32)]).reshape(_NW * ch, _CB)

    colsrc = [0] * d
    for g in range(d // (2 * _L)):
        for j in range(_L):
            colsrc[g * 2 * _L + 2 * j] = g * 2 * _L + j
            colsrc[g * 2 * _L + 2 * j + 1] = g * 2 * _L + _L + j
    perm = jnp.array(colsrc, dtype=jnp.int32)
    W1_l_sc = W1_l[perm]
    W2_l_sc = W2_l[perm]

    bn = max((b for b in range(16, min(n, 1024) + 1, 16) if n % b == 0),
             default=n)
    grid = (n // bn,)
    row_spec = pl.BlockSpec((bn, d), lambda i: (i, 0))
    w_spec = pl.BlockSpec((d, d), lambda i: (0, 0))
    b_spec = pl.BlockSpec((1, d), lambda i: (0, 0))
    part0_spec = pl.BlockSpec((1, bn, d), lambda i: (0, i, 0))
    part1_spec = pl.BlockSpec((1, bn, d), lambda i: (1, i, 0))
    inv_spec = pl.BlockSpec((bn, 1), lambda i: (i, 0))
    row_ty = jax.ShapeDtypeStruct((n, d), _F32)

    y1l, y1r = pl.pallas_call(
        _lin2_kernel, grid=grid,
        in_specs=[row_spec, w_spec, w_spec, b_spec],
        out_specs=[row_spec, row_spec],
        out_shape=[row_ty, row_ty],
    )(x, W1_l_sc, W1_r, b1.reshape(1, d))

    part1, cnt = _sc_aggregate(y1l.astype(jnp.bfloat16), src_rs, dst_rs,
                               n_acc, True)

    inv = pl.pallas_call(
        _inv_kernel, grid=(n_acc // _CB,),
        in_specs=[pl.BlockSpec((_NW, _CB), lambda i: (0, i))],
        out_specs=pl.BlockSpec((_CB, 1), lambda i: (i, 0)),
        out_shape=jax.ShapeDtypeStruct((n_acc, 1), _F32),
    )(cnt)

    y2l, y2r = pl.pallas_call(
        _mid_kernel, grid=grid,
        in_specs=[part0_spec, part1_spec, inv_spec, row_spec,
                  w_spec, w_spec, b_spec],
        out_specs=[row_spec, row_spec],
        out_shape=[row_ty, row_ty],
    )(part1, part1, inv, y1r, W2_l_sc, W2_r, b2.reshape(1, d))

    part2 = _sc_aggregate(y2l.astype(jnp.bfloat16), src_rs, dst_rs,
                          n_acc, False)[0]

    out = pl.pallas_call(
        _final_kernel, grid=grid,
        in_specs=[part0_spec, part1_spec, inv_spec, row_spec],
        out_specs=row_spec,
        out_shape=row_ty,
    )(part2, part2, inv, y2r)
    return out


# speedup vs baseline: 6.0343x; 1.0624x over previous
import functools
import math

import jax
import jax.numpy as jnp
from jax import lax
from jax.experimental import pallas as pl
from jax.experimental.pallas import tpu as pltpu
from jax.experimental.pallas import tpu_sc as plsc

_NC = 2
_NS = 16
_L = 16
_NW = _NC * _NS
_CB = 128

_F32 = jnp.float32


def _dot_t(a, w):
    return lax.dot_general(a, w, (((1,), (1,)), ((), ())),
                           preferred_element_type=_F32,
                           precision=lax.Precision.HIGHEST)


def _lin2_kernel(x_ref, wl_ref, wr_ref, b_ref, yl_ref, yr_ref):
    xb = x_ref[...]
    yl_ref[...] = _dot_t(xb, wl_ref[...])
    yr_ref[...] = _dot_t(xb, wr_ref[...]) + b_ref[...]


def _inv_kernel(cnt_ref, inv_ref):
    c = jnp.sum(cnt_ref[...], axis=0)
    inv_ref[...] = (1.0 / jnp.maximum(c, 1.0))[:, None]


def _mid_kernel(p0_ref, p1_ref, inv_ref, yr_ref, wl_ref, wr_ref, b_ref,
                y2l_ref, y2r_ref):
    h = (p0_ref[0] + p1_ref[0]) * inv_ref[...] + yr_ref[...]
    h = jnp.maximum(h, 0.0)
    y2l_ref[...] = _dot_t(h, wl_ref[...])
    y2r_ref[...] = _dot_t(h, wr_ref[...]) + b_ref[...]


def _final_kernel(q0_ref, q1_ref, inv_ref, yr_ref, out_ref):
    out_ref[...] = (q0_ref[0] + q1_ref[0]) * inv_ref[...] + yr_ref[...]


def _sc_aggregate(y, src_rs, dst_rs, n_acc, with_counts):
    n, d = y.shape
    ch = src_rs.shape[0] // _NW
    nzc = n_acc // _CB
    zc_max = -(-nzc // _NS)

    mesh = plsc.VectorSubcoreMesh(core_axis_name="c", subcore_axis_name="s",
                                  num_cores=_NC, num_subcores=_NS)
    out_type = (jax.ShapeDtypeStruct((_NC, n_acc, d), _F32),)
    if with_counts:
        out_type += (jax.ShapeDtypeStruct((_NW, n_acc), _F32),)
    scratch = [
        pltpu.VMEM_SHARED((n_acc, d), _F32),
        pltpu.VMEM((4, _CB), jnp.int32),
        pltpu.VMEM((4, _CB), jnp.int32),
        pltpu.VMEM((_CB, d), jnp.bfloat16),
        pltpu.VMEM((_CB, d), jnp.bfloat16),
        pltpu.VMEM((_CB, d), _F32),
        pltpu.VMEM((n_acc if with_counts else _L,), _F32),
        pltpu.SemaphoreType.DMA,
        pltpu.SemaphoreType.DMA,
        pltpu.SemaphoreType.DMA,
        pltpu.SemaphoreType.DMA,
        pltpu.SemaphoreType.DMA,
        pltpu.SemaphoreType.DMA,
        pltpu.SemaphoreType.DMA,
        pltpu.SemaphoreType.DMA,
    ]

    def body(y_hbm, src_hbm, dst_hbm, *rest):
        if with_counts:
            part_hbm, cnt_hbm = rest[0], rest[1]
            rest = rest[2:]
        else:
            part_hbm = rest[0]
            rest = rest[1:]
        (acc, src_i, dst_i, rows0, rows1, rowsf, cnt_v,
         sg0, sg1, ss0, ss1, si0, si1, si2, si3) = rest
        cid = lax.axis_index("c")
        sid = lax.axis_index("s")
        wid = cid * _NS + sid
        zero = jnp.zeros((_L,), _F32)
        ones = jnp.ones((_L,), _F32)
        rows = (rows0, rows1)
        sg = (sg0, sg1)
        ss = (ss0, ss1)
        si = (si0, si1, si2, si3)
        base = wid * ch

        def stage_idx(g, s):
            pltpu.async_copy(src_hbm.at[base + g], src_i.at[s], si[s])
            pltpu.async_copy(dst_hbm.at[base + g], dst_i.at[s], si[s])

        def wait_idx(g, s):
            pltpu.make_async_copy(src_hbm.at[base + g], src_i.at[s],
                                  si[s]).wait()
            pltpu.make_async_copy(dst_hbm.at[base + g], dst_i.at[s],
                                  si[s]).wait()

        for s in range(4):
            stage_idx(s, s)

        def _zrow(i, _):
            for j in range(d // _L):
                rowsf[i, pl.ds(j * _L, _L)] = zero
            return 0
        lax.fori_loop(0, _CB, _zrow, 0)
        for k in range(zc_max):
            c = sid + k * _NS
            if (k + 1) * _NS <= nzc:
                pltpu.sync_copy(rowsf, acc.at[pl.ds(c * _CB, _CB)])
            else:
                @pl.when(c < nzc)
                def _():
                    pltpu.sync_copy(rowsf, acc.at[pl.ds(c * _CB, _CB)])
        if with_counts:
            def _zcnt(i, _):
                cnt_v[pl.ds(i * _L, _L)] = zero
                return 0
            lax.fori_loop(0, n_acc // _L, _zcnt, 0)
        plsc.subcore_barrier()

        wait_idx(0, 0)
        pltpu.async_copy(y_hbm.at[src_i.at[0]], rows0, sg0)
        wait_idx(1, 1)
        pltpu.async_copy(y_hbm.at[src_i.at[1]], rows1, sg1)

        def _chunk(g, s, b):
            pltpu.make_async_copy(y_hbm.at[src_i.at[s]], rows[b],
                                  sg[b]).wait()
            @plsc.parallel_loop(0, _CB, unroll=8)
            def _conv(i):
                for j in range(d // (2 * _L)):
                    ab = rows[b][i, pl.ds(j * 2 * _L, 2 * _L)]
                    lo, hi = plsc.unpack(ab, format=plsc.PackFormat.INTERLEAVED)
                    rowsf[i, pl.ds(j * 2 * _L, _L)] = lo
                    rowsf[i, pl.ds(j * 2 * _L + _L, _L)] = hi
            @pl.when(g + 2 < ch)
            def _():
                s2 = (s + 2) % 4
                wait_idx(g + 2, s2)
                pltpu.async_copy(y_hbm.at[src_i.at[s2]], rows[b], sg[b])
            sc_desc = pltpu.async_copy(rowsf, acc.at[dst_i.at[s]], ss[b],
                                       add=True)
            if with_counts:
                for j in range(_CB // _L):
                    ids = dst_i[s, pl.ds(j * _L, _L)]
                    plsc.addupdate_scatter(cnt_v, [ids], ones)
            sc_desc.wait()
            @pl.when(g + 4 < ch)
            def _():
                stage_idx(g + 4, s)

        def _iter(i, _):
            g = 4 * i
            _chunk(g, 0, 0)
            _chunk(g + 1, 1, 1)
            _chunk(g + 2, 2, 0)
            _chunk(g + 3, 3, 1)
            return 0
        lax.fori_loop(0, ch // 4, _iter, 0)

        plsc.subcore_barrier()
        for k in range(zc_max):
            c = sid + k * _NS
            if (k + 1) * _NS <= nzc:
                pltpu.sync_copy(acc.at[pl.ds(c * _CB, _CB)],
                                part_hbm.at[cid, pl.ds(c * _CB, _CB)])
            else:
                @pl.when(c < nzc)
                def _():
                    pltpu.sync_copy(acc.at[pl.ds(c * _CB, _CB)],
                                    part_hbm.at[cid, pl.ds(c * _CB, _CB)])
        if with_counts:
            pltpu.sync_copy(cnt_v, cnt_hbm.at[wid])

    params = pltpu.CompilerParams(needs_layout_passes=False,
                                  use_tc_tiling_on_sc=False)
    return pl.kernel(body, out_type=out_type, mesh=mesh,
                     scratch_types=scratch,
                     compiler_params=params)(y, src_rs, dst_rs)


def kernel(x, edge_index, W1_l, W1_r, b1, W2_l, W2_r, b2):
    n, d = x.shape
    e = edge_index.shape[1]

    ch = -(-e // (_NW * _CB))
    ch = -(-ch // 4) * 4
    ep = _NW * _CB * ch
    n_acc = _CB * (-(-(n + 1) // _CB))

    src = edge_index[0].astype(jnp.int32)
    dst = edge_index[1].astype(jnp.int32)
    src_rs = jnp.concatenate(
        [src, jnp.zeros((ep - e,), jnp.int32)]).reshape(_NW * ch, _CB)
    dst_rs = jnp.concatenate(
        [dst, jnp.full((ep - e,), n, jnp.int32)]).reshape(_NW * ch, _CB)

    colsrc = [0] * d
    for g in range(d // (2 * _L)):
        for j in range(_L):
            colsrc[g * 2 * _L + 2 * j] = g * 2 * _L + j
            colsrc[g * 2 * _L + 2 * j + 1] = g * 2 * _L + _L + j
    perm = jnp.array(colsrc, dtype=jnp.int32)
    W1_l_sc = W1_l[perm]
    W2_l_sc = W2_l[perm]

    bn = max((b for b in range(16, min(n, 1024) + 1, 16) if n % b == 0),
             default=n)
    grid = (n // bn,)
    row_spec = pl.BlockSpec((bn, d), lambda i: (i, 0))
    w_spec = pl.BlockSpec((d, d), lambda i: (0, 0))
    b_spec = pl.BlockSpec((1, d), lambda i: (0, 0))
    part0_spec = pl.BlockSpec((1, bn, d), lambda i: (0, i, 0))
    part1_spec = pl.BlockSpec((1, bn, d), lambda i: (1, i, 0))
    inv_spec = pl.BlockSpec((bn, 1), lambda i: (i, 0))
    row_ty = jax.ShapeDtypeStruct((n, d), _F32)

    y1l, y1r = pl.pallas_call(
        _lin2_kernel, grid=grid,
        in_specs=[row_spec, w_spec, w_spec, b_spec],
        out_specs=[row_spec, row_spec],
        out_shape=[row_ty, row_ty],
    )(x, W1_l_sc, W1_r, b1.reshape(1, d))

    part1, cnt = _sc_aggregate(y1l.astype(jnp.bfloat16), src_rs, dst_rs,
                               n_acc, True)

    inv = pl.pallas_call(
        _inv_kernel, grid=(n_acc // _CB,),
        in_specs=[pl.BlockSpec((_NW, _CB), lambda i: (0, i))],
        out_specs=pl.BlockSpec((_CB, 1), lambda i: (i, 0)),
        out_shape=jax.ShapeDtypeStruct((n_acc, 1), _F32),
    )(cnt)

    y2l, y2r = pl.pallas_call(
        _mid_kernel, grid=grid,
        in_specs=[part0_spec, part1_spec, inv_spec, row_spec,
                  w_spec, w_spec, b_spec],
        out_specs=[row_spec, row_spec],
        out_shape=[row_ty, row_ty],
    )(part1, part1, inv, y1r, W2_l_sc, W2_r, b2.reshape(1, d))

    part2 = _sc_aggregate(y2l.astype(jnp.bfloat16), src_rs, dst_rs,
                          n_acc, False)[0]

    out = pl.pallas_call(
        _final_kernel, grid=grid,
        in_specs=[part0_spec, part1_spec, inv_spec, row_spec],
        out_specs=row_spec,
        out_shape=row_ty,
    )(part2, part2, inv, y2r)
    return out
